# Initial kernel scaffold; baseline (speedup 1.0000x reference)
import jax
import jax.numpy as jnp
from jax.experimental import pallas as pl


def kernel(speakers, table):
    raise NotImplementedError("write your pallas kernel here")


# speedup vs baseline: 3.9377x; 3.9377x over previous
import functools

import jax
import jax.numpy as jnp
from jax import lax
from jax.experimental import pallas as pl
from jax.experimental.pallas import tpu as pltpu
from jax.experimental.pallas import tpu_sc as plsc

_EMBED = 64
_Q = 4
_QROW = _Q * _EMBED
_CHUNK = 256
_IDXW = 128


def _sc_lookup(planes, combo):
    nq = planes.shape[1]
    info = plsc.get_sparse_core_info()
    ncores, nsub = info.num_cores, info.num_subcores
    nw = ncores * nsub
    q_per_w = nq // nw
    n_chunks = q_per_w // _CHUNK
    mesh = plsc.VectorSubcoreMesh(core_axis_name="c", subcore_axis_name="s")

    @functools.partial(
        pl.kernel,
        mesh=mesh,
        out_type=jax.ShapeDtypeStruct((nq, _QROW), jnp.float32),
        scratch_types=[
            pltpu.VMEM((_Q, _CHUNK), jnp.int32),
            pltpu.VMEM((_CHUNK // _IDXW, _IDXW), jnp.int32),
            pltpu.VMEM((_CHUNK, _QROW), jnp.float32),
            pltpu.SemaphoreType.DMA,
        ],
    )
    def k(planes_hbm, combo_hbm, out_hbm, pidx_v, qidx_v, rows_v, sem):
        wid = lax.axis_index("s") * ncores + lax.axis_index("c")
        w_base = wid * q_per_w

        def body(i, carry):
            base = w_base + i * _CHUNK
            for r in range(_Q):
                pltpu.sync_copy(planes_hbm.at[r, pl.ds(base, _CHUNK)],
                                pidx_v.at[r])
            for v in range(_CHUNK // 16):
                sl = pl.ds(v * 16, 16)
                q = pidx_v[0, sl]
                for r in range(1, _Q):
                    q = q * 3 + pidx_v[r, sl]
                qidx_v[v // 8, pl.ds((v % 8) * 16, 16)] = q
            copies = [
                pltpu.async_copy(combo_hbm.at[qidx_v.at[j]],
                                 rows_v.at[pl.ds(j * _IDXW, _IDXW)], sem)
                for j in range(_CHUNK // _IDXW)
            ]
            for c in copies:
                c.wait()
            pltpu.sync_copy(rows_v, out_hbm.at[pl.ds(base, _CHUNK)])
            return carry

        lax.fori_loop(0, n_chunks, body, 0)

    return k(planes, combo)


def _quad_table(table):
    q = jnp.arange(81)
    rows = [table[(q // (3 ** (3 - k))) % 3] for k in range(_Q)]
    return jnp.concatenate(rows, axis=1)


def kernel(speakers, table):
    b, h = speakers.shape
    nq = (b * h) // _Q
    planes = speakers.reshape(nq, _Q).T.astype(jnp.int32)
    combo = _quad_table(table)
    out = _sc_lookup(planes, combo)
    return out.reshape(b, h, _EMBED)


# speedup vs baseline: 3.9721x; 1.0087x over previous
import functools

import jax
import jax.numpy as jnp
from jax import lax
from jax.experimental import pallas as pl
from jax.experimental.pallas import tpu as pltpu
from jax.experimental.pallas import tpu_sc as plsc

_EMBED = 64
_Q = 4
_QROW = _Q * _EMBED
_CHUNK = 128
_NBUF = 2


def _sc_lookup(planes, combo):
    nchunks_total = planes.shape[0]
    nq = nchunks_total * _CHUNK
    info = plsc.get_sparse_core_info()
    ncores, nsub = info.num_cores, info.num_subcores
    nw = ncores * nsub
    g_per_w = nchunks_total // nw
    n_outer = g_per_w // _NBUF
    mesh = plsc.VectorSubcoreMesh(core_axis_name="c", subcore_axis_name="s")

    @functools.partial(
        pl.kernel,
        mesh=mesh,
        out_type=jax.ShapeDtypeStruct((nq, _QROW), jnp.float32),
        scratch_types=[
            pltpu.VMEM((_NBUF, _Q, _CHUNK), jnp.int32),
            pltpu.VMEM((_NBUF, _CHUNK), jnp.int32),
            pltpu.VMEM((_NBUF, _CHUNK, _QROW), jnp.float32),
            pltpu.SemaphoreType.DMA,
            pltpu.SemaphoreType.DMA,
            pltpu.SemaphoreType.DMA,
            pltpu.SemaphoreType.DMA,
            pltpu.SemaphoreType.DMA,
            pltpu.SemaphoreType.DMA,
        ],
    )
    def k(planes_hbm, combo_hbm, out_hbm, pidx_v, qidx_v, rows_v,
          si0, si1, sg0, sg1, so0, so1):
        sem_i, sem_g, sem_o = (si0, si1), (sg0, sg1), (so0, so1)
        wid = lax.axis_index("s") * ncores + lax.axis_index("c")
        w_chunk0 = wid * g_per_w

        def fire_idx(g, b):
            pltpu.async_copy(planes_hbm.at[w_chunk0 + g], pidx_v.at[b],
                             sem_i[b])

        fire_idx(0, 0)
        fire_idx(1, 1)

        def body(it, carry):
            for b in range(_NBUF):
                g = it * _NBUF + b
                pltpu.make_async_copy(planes_hbm.at[w_chunk0 + g],
                                      pidx_v.at[b], sem_i[b]).wait()
                for v in range(_CHUNK // 16):
                    sl = pl.ds(v * 16, 16)
                    q = pidx_v[b, 0, sl]
                    for r in range(1, _Q):
                        q = q * 3 + pidx_v[b, r, sl]
                    qidx_v[b, sl] = q

                @pl.when(it < n_outer - 1)
                def _prefetch():
                    fire_idx(it * _NBUF + b + _NBUF, b)

                @pl.when(it >= 1)
                def _drain_out():
                    pltpu.make_async_copy(out_hbm.at[pl.ds(0, _CHUNK)],
                                          rows_v.at[b], sem_o[b]).wait()

                pltpu.async_copy(combo_hbm.at[qidx_v.at[b]], rows_v.at[b],
                                 sem_g[b]).wait()
                base = (w_chunk0 + g) * _CHUNK
                pltpu.async_copy(rows_v.at[b],
                                 out_hbm.at[pl.ds(base, _CHUNK)], sem_o[b])
            return carry

        lax.fori_loop(0, n_outer, body, 0)
        for b in range(_NBUF):
            pltpu.make_async_copy(out_hbm.at[pl.ds(0, _CHUNK)],
                                  rows_v.at[b], sem_o[b]).wait()

    return k(planes, combo)


def _quad_table(table):
    q = jnp.arange(81)
    rows = [table[(q // (3 ** (3 - k))) % 3] for k in range(_Q)]
    return jnp.concatenate(rows, axis=1)


def kernel(speakers, table):
    b, h = speakers.shape
    nq = (b * h) // _Q
    nchunks = nq // _CHUNK
    planes = (speakers.reshape(nchunks, _CHUNK, _Q)
              .transpose(0, 2, 1)
              .astype(jnp.int32))
    combo = _quad_table(table)
    out = _sc_lookup(planes, combo)
    return out.reshape(b, h, _EMBED)
